# Initial kernel scaffold; baseline (speedup 1.0000x reference)
import jax
import jax.numpy as jnp
from jax.experimental import pallas as pl


def kernel(w1p, b1p, w2cat, b2p, wf1r, bf1p, wf2p, bf2p, wf3p, bf3p, x):
    raise NotImplementedError("write your pallas kernel here")


# speedup vs baseline: 90.3511x; 90.3511x over previous
import jax
import jax.numpy as jnp
from jax import lax
from jax.experimental import pallas as pl
from jax.experimental.pallas import tpu as pltpu

_BB = 256


def _fused_lenet_kernel(x_ref, w1_ref, w2_ref, wf1_ref, wf2_ref, wf3_ref,
                        o_ref, a1_scr):
    f32 = jnp.float32
    ones1 = jnp.ones((1, _BB), f32)
    extra4 = jnp.concatenate([ones1, jnp.zeros((3, _BB), f32)], axis=0)
    extra6 = jnp.concatenate([ones1, jnp.zeros((5, _BB), f32)], axis=0)
    extra8 = jnp.concatenate([ones1, jnp.zeros((7, _BB), f32)], axis=0)

    w1 = w1_ref[...]
    for p in range(12):
        cands = []
        for dh in (0, 1):
            ho = 2 * p + dh
            rows = jnp.concatenate(
                [x_ref[pl.ds(28 * ho, 140), :], extra4], axis=0)
            cands.append(jnp.dot(w1, rows, preferred_element_type=f32))
        m = jnp.maximum(cands[0], cands[1]).reshape(12, 40, _BB)
        m = jnp.maximum(m[:, :20, :], m[:, 20:, :])
        a1_scr[pl.ds(240 * p, 240), :] = jnp.maximum(m, 0.0).reshape(240, _BB)

    w2 = w2_ref[...]
    a2_parts = []
    for p in range(5):
        cands = []
        for dh in (0, 1):
            ho = 2 * p + dh
            rows = jnp.concatenate(
                [a1_scr[pl.ds(240 * ho, 720), :], extra8], axis=0)
            cands.append(jnp.dot(w2, rows, preferred_element_type=f32))
        m = jnp.maximum(cands[0], cands[1])[:500].reshape(5, 100, _BB)
        m = jnp.maximum(m[:, :50, :], m[:, 50:, :])
        a2_parts.append(jnp.maximum(m, 0.0).reshape(250, _BB))

    a2 = jnp.concatenate(a2_parts + [extra6], axis=0)
    h1 = jnp.maximum(
        jnp.dot(wf1_ref[...], a2, preferred_element_type=f32), 0.0)
    h2 = jnp.maximum(
        jnp.dot(wf2_ref[...], jnp.concatenate([h1, extra8], axis=0),
                preferred_element_type=f32), 0.0)
    o_ref[...] = jnp.dot(
        wf3_ref[...], jnp.concatenate([h2, extra8], axis=0),
        preferred_element_type=f32)


def _toeplitz_weights(w1p, b1p, w2cat, b2p, wf1r, bf1p, wf2p, bf2p, wf3p, bf3p):
    f32 = jnp.float32
    w1 = w1p[:, :20].reshape(5, 5, 20)
    t1 = jnp.stack([jnp.pad(w1, ((0, 0), (wo, 23 - wo), (0, 0)))
                    for wo in range(24)])
    t1 = t1.transpose(0, 3, 1, 2).reshape(480, 140)
    b1c = jnp.tile(b1p[0, :20], 24).reshape(480, 1)
    w1t = jnp.concatenate([t1, b1c, jnp.zeros((480, 3), f32)], axis=1)

    w2 = w2cat.reshape(9, 128, 128)[:, :20, :50].reshape(3, 3, 20, 50)
    t2 = jnp.stack([jnp.pad(w2, ((0, 0), (wo, 9 - wo), (0, 0), (0, 0)))
                    for wo in range(10)])
    t2 = t2.transpose(0, 4, 1, 2, 3).reshape(500, 720)
    b2c = jnp.tile(b2p[0, :50], 10).reshape(500, 1)
    w2t = jnp.concatenate([t2, b2c, jnp.zeros((500, 7), f32)], axis=1)
    w2t = jnp.concatenate([w2t, jnp.zeros((4, 728), f32)], axis=0)

    wf1 = wf1r.reshape(5, 5, 128, 256)[:, :, :50, :].reshape(1250, 256).T
    wf1t = jnp.concatenate(
        [wf1, bf1p.T, jnp.zeros((256, 5), f32)], axis=1)
    wf2t = jnp.concatenate([wf2p.T, bf2p.T, jnp.zeros((128, 7), f32)], axis=1)
    wf3t = jnp.concatenate([wf3p.T, bf3p.T, jnp.zeros((128, 7), f32)], axis=1)
    return w1t, w2t, wf1t, wf2t, wf3t


def kernel(w1p, b1p, w2cat, b2p, wf1r, bf1p, wf2p, bf2p, wf3p, bf3p, x):
    B = x.shape[0]
    w1t, w2t, wf1t, wf2t, wf3t = _toeplitz_weights(
        w1p, b1p, w2cat, b2p, wf1r, bf1p, wf2p, bf2p, wf3p, bf3p)
    xt = x.astype(jnp.float32).reshape(B, 784).T

    steps = B // _BB
    logits = pl.pallas_call(
        _fused_lenet_kernel,
        out_shape=jax.ShapeDtypeStruct((128, B), jnp.float32),
        grid=(steps,),
        in_specs=[
            pl.BlockSpec((784, _BB), lambda b: (0, b)),
            pl.BlockSpec((480, 144), lambda b: (0, 0)),
            pl.BlockSpec((504, 728), lambda b: (0, 0)),
            pl.BlockSpec((256, 1256), lambda b: (0, 0)),
            pl.BlockSpec((128, 264), lambda b: (0, 0)),
            pl.BlockSpec((128, 136), lambda b: (0, 0)),
        ],
        out_specs=pl.BlockSpec((128, _BB), lambda b: (0, b)),
        scratch_shapes=[pltpu.VMEM((2880, _BB), jnp.float32)],
        compiler_params=pltpu.CompilerParams(
            dimension_semantics=("parallel",),
            vmem_limit_bytes=64 * 1024 * 1024,
        ),
    )(xt, w1t, w2t, wf1t, wf2t, wf3t)
    return logits[:10, :].T
